# baseline (speedup 1.0000x reference)
import functools

import jax
import jax.numpy as jnp
from jax import lax
from jax.experimental import pallas as pl
from jax.experimental.pallas import tpu as pltpu
from jax.experimental.pallas import tpu_sc as plsc

NUM_EMBEDDINGS = 100000
PACKED_WORDS = 16
EMB_DIM = 128
ROWS = 16384 * 26
NC, NS, L = 2, 16, 16
NW = NC * NS
ROWS_PER_W = ROWS // NW
CHUNK = 128
NCHUNK = ROWS_PER_W // CHUNK


def _make_kernel():
  mesh = plsc.VectorSubcoreMesh(core_axis_name="c", subcore_axis_name="s")

  @functools.partial(
      pl.kernel,
      mesh=mesh,
      out_type=jax.ShapeDtypeStruct((ROWS, EMB_DIM), jnp.float32),
      compiler_params=pltpu.CompilerParams(use_tc_tiling_on_sc=False),
      scratch_types=[
          pltpu.VMEM((ROWS_PER_W,), jnp.int32),
          pltpu.VMEM((CHUNK, PACKED_WORDS), jnp.int32),
          pltpu.VMEM((CHUNK, PACKED_WORDS), jnp.int32),
          pltpu.VMEM((CHUNK, EMB_DIM), jnp.float32),
          pltpu.VMEM((CHUNK, EMB_DIM), jnp.float32),
          pltpu.VMEM((L,), jnp.float32),
          pltpu.SemaphoreType.DMA,
          pltpu.SemaphoreType.DMA,
          pltpu.SemaphoreType.DMA,
          pltpu.SemaphoreType.DMA,
      ],
  )
  def k(tab_hbm, idx_hbm, c_hbm, out_hbm, idx_all, rows0, rows1, out0, out1,
        c_v, sg0, sg1, so0, so1):
    wid = lax.axis_index("s") * NC + lax.axis_index("c")
    tbase = wid * ROWS_PER_W

    rows = (rows0, rows1)
    outs = (out0, out1)
    sg = (sg0, sg1)
    so = (so0, so1)

    pltpu.sync_copy(idx_hbm.at[pl.ds(tbase, ROWS_PER_W)], idx_all)
    pltpu.sync_copy(c_hbm, c_v)
    c_vec = c_v[...]
    lut = (lax.iota(jnp.int32, L).astype(jnp.float32) - 7.0) / c_vec

    def vperm(src, idx):
      return lax.gather(
          src, idx[:, None],
          lax.GatherDimensionNumbers(
              offset_dims=(), collapsed_slice_dims=(0,),
              start_index_map=(0,)),
          slice_sizes=(1,),
          mode=lax.GatherScatterMode.PROMISE_IN_BOUNDS)

    def start_gather(g, b):
      pltpu.async_copy(
          tab_hbm.at[idx_all.at[pl.ds(g * CHUNK, CHUNK)]], rows[b], sg[b])

    def wait_gather(b):
      pltpu.make_async_copy(
          tab_hbm.at[pl.ds(0, CHUNK), :], rows[b], sg[b]).wait()

    def start_out(g, b):
      pltpu.async_copy(
          outs[b], out_hbm.at[pl.ds(tbase + g * CHUNK, CHUNK), :], so[b])

    def wait_out(b):
      pltpu.make_async_copy(
          outs[b], out_hbm.at[pl.ds(0, CHUNK), :], so[b]).wait()

    start_gather(0, 0)

    def chunk_pair(g2, _):
      for b in range(2):
        g = 2 * g2 + b
        nxt = g + 1

        @pl.when(nxt < NCHUNK)
        def _():
          start_gather(nxt, 1 - b)

        wait_gather(b)

        @pl.when(g >= 2)
        def _():
          wait_out(b)

        rows_b = rows[b]
        out_b = outs[b]

        def row_body(i, _):
          w = rows_b[i, :]
          for s in range(8):
            nib = lax.shift_right_logical(w, 4 * s) & 15
            out_b[i, pl.ds(s * L, L)] = vperm(lut, nib)
          return 0

        lax.fori_loop(0, CHUNK, row_body, 0, unroll=4)
        start_out(g, b)
      return 0

    lax.fori_loop(0, NCHUNK // 2, chunk_pair, 0)
    wait_out(0)
    wait_out(1)

  return k


_sc_kernel = _make_kernel()


@jax.jit
def kernel(x, weight_quant_packed, c):
  hi = (weight_quant_packed >> 4).astype(jnp.uint32)
  lo = (weight_quant_packed & 15).astype(jnp.uint32)
  nibs = jnp.stack([hi, lo], axis=-1).reshape(NUM_EMBEDDINGS, 8, PACKED_WORDS)
  shifts = (4 * jnp.arange(8, dtype=jnp.uint32))[None, :, None]
  tab32 = lax.bitcast_convert_type(
      (nibs << shifts).sum(axis=1, dtype=jnp.uint32), jnp.int32)
  idx = x.reshape(ROWS)
  c_vec = jnp.full((L,), c, dtype=jnp.float32)
  out = _sc_kernel(tab32, idx, c_vec)
  return out.reshape(x.shape + (EMB_DIM,))


# speedup vs baseline: 1.4561x; 1.4561x over previous
import functools

import jax
import jax.numpy as jnp
from jax import lax
from jax.experimental import pallas as pl
from jax.experimental.pallas import tpu as pltpu
from jax.experimental.pallas import tpu_sc as plsc

NUM_EMBEDDINGS = 100000
PACKED_WORDS = 16
EMB_DIM = 128
BATCH = 16384
FIELDS = 26
ROWS = BATCH * FIELDS
NC, NS, L = 2, 16, 16
NW = NC * NS
ROWS_PER_W = ROWS // NW
CHUNK = 128
NCHUNK = ROWS_PER_W // CHUNK

ELS_PER_W = BATCH // NW
CH_ELS = 4
CH_ROWS = CH_ELS * FIELDS
CH_WORDS = CH_ROWS * PACKED_WORDS
NCHUNK2 = ELS_PER_W // CH_ELS


def _make_gather_kernel():
  mesh = plsc.VectorSubcoreMesh(core_axis_name="c", subcore_axis_name="s")

  @functools.partial(
      pl.kernel,
      mesh=mesh,
      out_type=jax.ShapeDtypeStruct((ROWS, PACKED_WORDS), jnp.int32),
      compiler_params=pltpu.CompilerParams(use_tc_tiling_on_sc=False),
      scratch_types=[
          pltpu.VMEM((ROWS_PER_W,), jnp.int32),
          pltpu.VMEM((CHUNK, PACKED_WORDS), jnp.int32),
          pltpu.VMEM((CHUNK, PACKED_WORDS), jnp.int32),
          pltpu.SemaphoreType.DMA,
          pltpu.SemaphoreType.DMA,
          pltpu.SemaphoreType.DMA,
          pltpu.SemaphoreType.DMA,
      ],
  )
  def k(tab_hbm, idx_hbm, g_hbm, idx_all, rows0, rows1, sg0, sg1, so0, so1):
    wid = lax.axis_index("s") * NC + lax.axis_index("c")
    tbase = wid * ROWS_PER_W

    rows = (rows0, rows1)
    sg = (sg0, sg1)
    so = (so0, so1)

    pltpu.sync_copy(idx_hbm.at[pl.ds(tbase, ROWS_PER_W)], idx_all)

    def start_gather(g, b):
      pltpu.async_copy(
          tab_hbm.at[idx_all.at[pl.ds(g * CHUNK, CHUNK)]], rows[b], sg[b])

    def wait_gather(b):
      pltpu.make_async_copy(
          tab_hbm.at[pl.ds(0, CHUNK), :], rows[b], sg[b]).wait()

    def start_out(g, b):
      pltpu.async_copy(
          rows[b], g_hbm.at[pl.ds(tbase + g * CHUNK, CHUNK), :], so[b])

    def wait_out(b):
      pltpu.make_async_copy(
          rows[b], g_hbm.at[pl.ds(0, CHUNK), :], so[b]).wait()

    start_gather(0, 0)

    def chunk_pair(g2, _):
      for b in range(2):
        g = 2 * g2 + b
        nxt = g + 1
        wait_gather(b)

        @pl.when(g >= 2)
        def _():
          wait_out(b)

        start_out(g, b)

        @pl.when(nxt < NCHUNK)
        def _():
          start_gather(nxt, 1 - b)
      return 0

    lax.fori_loop(0, NCHUNK // 2, chunk_pair, 0)
    wait_out(0)
    wait_out(1)

  return k


def _make_dequant_kernel():
  mesh = plsc.VectorSubcoreMesh(core_axis_name="c", subcore_axis_name="s")

  @functools.partial(
      pl.kernel,
      mesh=mesh,
      out_type=jax.ShapeDtypeStruct((BATCH, FIELDS, EMB_DIM), jnp.float32),
      compiler_params=pltpu.CompilerParams(use_tc_tiling_on_sc=True),
      scratch_types=[
          pltpu.VMEM((CH_WORDS,), jnp.int32),
          pltpu.VMEM((CH_WORDS,), jnp.int32),
          pltpu.VMEM((CH_ROWS, EMB_DIM), jnp.float32),
          pltpu.VMEM((CH_ROWS, EMB_DIM), jnp.float32),
          pltpu.VMEM((L,), jnp.float32),
          pltpu.SemaphoreType.DMA,
          pltpu.SemaphoreType.DMA,
          pltpu.SemaphoreType.DMA,
          pltpu.SemaphoreType.DMA,
      ],
  )
  def k(g_hbm, c_hbm, out_hbm, gv0, gv1, ov0, ov1, c_v, sg0, sg1, so0, so1):
    wid = lax.axis_index("s") * NC + lax.axis_index("c")
    ebase = wid * ELS_PER_W

    gv = (gv0, gv1)
    ov = (ov0, ov1)
    sg = (sg0, sg1)
    so = (so0, so1)

    pltpu.sync_copy(c_hbm, c_v)
    lut = (lax.iota(jnp.int32, L).astype(jnp.float32) - 7.0) / c_v[...]

    def vperm(src, idx):
      return lax.gather(
          src, idx[:, None],
          lax.GatherDimensionNumbers(
              offset_dims=(), collapsed_slice_dims=(0,),
              start_index_map=(0,)),
          slice_sizes=(1,),
          mode=lax.GatherScatterMode.PROMISE_IN_BOUNDS)

    def start_load(g, b):
      off = (ebase + g * CH_ELS) * FIELDS * PACKED_WORDS
      pltpu.async_copy(g_hbm.at[pl.ds(off, CH_WORDS)], gv[b], sg[b])

    def wait_load(b):
      pltpu.make_async_copy(g_hbm.at[pl.ds(0, CH_WORDS)], gv[b], sg[b]).wait()

    def start_out(g, b):
      b0 = ebase + g * CH_ELS
      for e in range(CH_ELS):
        pltpu.async_copy(
            ov[b].at[pl.ds(e * FIELDS, FIELDS), :], out_hbm.at[b0 + e], so[b])

    def wait_out(b):
      for e in range(CH_ELS):
        pltpu.make_async_copy(
            ov[b].at[pl.ds(e * FIELDS, FIELDS), :], out_hbm.at[0], so[b]
        ).wait()

    start_load(0, 0)

    def chunk_pair(g2, _):
      for b in range(2):
        g = 2 * g2 + b
        nxt = g + 1

        @pl.when(nxt < NCHUNK2)
        def _():
          start_load(nxt, 1 - b)

        wait_load(b)

        @pl.when(g >= 2)
        def _():
          wait_out(b)

        gvb = gv[b]
        ovb = ov[b]

        def row_body(q, _):
          w = gvb[pl.ds(q * PACKED_WORDS, PACKED_WORDS)]
          for s in range(8):
            nib = lax.shift_right_logical(w, 4 * s) & 15
            ovb[q, pl.ds(s * L, L)] = vperm(lut, nib)
          return 0

        lax.fori_loop(0, CH_ROWS, row_body, 0, unroll=4)
        start_out(g, b)
      return 0

    lax.fori_loop(0, NCHUNK2 // 2, chunk_pair, 0)
    wait_out(0)
    wait_out(1)

  return k


_gather_kernel = _make_gather_kernel()
_dequant_kernel = _make_dequant_kernel()


def _permute_table(wqp):
  hi = wqp >> 4
  lo = wqp & 15

  def nib_plane(s):
    h = hi[:, 8 * s:8 * s + 8]
    l = lo[:, 8 * s:8 * s + 8]
    return jnp.stack([h, l], axis=-1).reshape(NUM_EMBEDDINGS, 16)

  byte_cols = [nib_plane(2 * kk) | (nib_plane(2 * kk + 1) << 4)
               for kk in range(4)]
  ptb = jnp.stack(byte_cols, axis=-1)
  return lax.bitcast_convert_type(ptb, jnp.int32)


@jax.jit
def kernel(x, weight_quant_packed, c):
  tab32 = _permute_table(weight_quant_packed)
  idx = x.reshape(ROWS)
  c_vec = jnp.full((L,), c, dtype=jnp.float32)
  g = _gather_kernel(tab32, idx)
  return _dequant_kernel(g.reshape(ROWS * PACKED_WORDS), c_vec)


# speedup vs baseline: 1.7141x; 1.1772x over previous
import functools

import jax
import jax.numpy as jnp
from jax import lax
from jax.experimental import pallas as pl
from jax.experimental.pallas import tpu as pltpu
from jax.experimental.pallas import tpu_sc as plsc

NUM_EMBEDDINGS = 100000
PACKED_WORDS = 16
EMB_DIM = 128
BATCH = 16384
FIELDS = 26
ROWS = BATCH * FIELDS
NC, NS, L = 2, 16, 16
NW = NC * NS
ROWS_PER_W = ROWS // NW
CHUNK = 128
NCHUNK = ROWS_PER_W // CHUNK
CHUNK_WORDS = CHUNK * PACKED_WORDS


def _make_gather_kernel():
  mesh = plsc.VectorSubcoreMesh(core_axis_name="c", subcore_axis_name="s")

  @functools.partial(
      pl.kernel,
      mesh=mesh,
      out_type=jax.ShapeDtypeStruct((ROWS, PACKED_WORDS), jnp.int32),
      compiler_params=pltpu.CompilerParams(use_tc_tiling_on_sc=False),
      scratch_types=[
          pltpu.VMEM((ROWS_PER_W,), jnp.int32),
          pltpu.VMEM((CHUNK, PACKED_WORDS), jnp.int32),
          pltpu.VMEM((CHUNK, PACKED_WORDS), jnp.int32),
          pltpu.VMEM((CHUNK, PACKED_WORDS), jnp.int32),
          pltpu.VMEM((CHUNK, PACKED_WORDS), jnp.int32),
          pltpu.SemaphoreType.DMA,
          pltpu.SemaphoreType.DMA,
          pltpu.SemaphoreType.DMA,
          pltpu.SemaphoreType.DMA,
      ],
  )
  def k(tab_hbm, idx_hbm, g_hbm, idx_all, rows0, rows1, pr0, pr1,
        sg0, sg1, so0, so1):
    wid = lax.axis_index("s") * NC + lax.axis_index("c")
    tbase = wid * ROWS_PER_W

    rows = (rows0, rows1)
    prs = (pr0, pr1)
    sg = (sg0, sg1)
    so = (so0, so1)

    pltpu.sync_copy(idx_hbm.at[pl.ds(tbase, ROWS_PER_W)], idx_all)

    it = lax.iota(jnp.int32, L)
    wordsel = it >> 3
    shvec = ((it >> 1) & 3) * 8 + (1 - (it & 1)) * 4

    def vperm(src, idx):
      return lax.gather(
          src, idx[:, None],
          lax.GatherDimensionNumbers(
              offset_dims=(), collapsed_slice_dims=(0,),
              start_index_map=(0,)),
          slice_sizes=(1,),
          mode=lax.GatherScatterMode.PROMISE_IN_BOUNDS)

    def start_gather(g, b):
      pltpu.async_copy(
          tab_hbm.at[idx_all.at[pl.ds(g * CHUNK, CHUNK)]], rows[b], sg[b])

    def wait_gather(b):
      pltpu.make_async_copy(
          tab_hbm.at[pl.ds(0, CHUNK), :], rows[b], sg[b]).wait()

    def start_out(g, b):
      pltpu.async_copy(
          prs[b], g_hbm.at[pl.ds(tbase + g * CHUNK, CHUNK), :], so[b])

    def wait_out(b):
      pltpu.make_async_copy(
          prs[b], g_hbm.at[pl.ds(0, CHUNK), :], so[b]).wait()

    start_gather(0, 0)

    def chunk_pair(g2, _):
      for b in range(2):
        g = 2 * g2 + b
        nxt = g + 1

        @pl.when(nxt < NCHUNK)
        def _():
          start_gather(nxt, 1 - b)

        wait_gather(b)

        @pl.when(g >= 2)
        def _():
          wait_out(b)

        rows_b = rows[b]
        pr_b = prs[b]

        def row_body(i, _):
          w = rows_b[i, :]
          acc = (lax.shift_right_logical(vperm(w, wordsel), shvec) & 15)
          for s in range(1, 8):
            nib = lax.shift_right_logical(vperm(w, wordsel + 2 * s), shvec) & 15
            acc = acc | (nib << (4 * s))
          pr_b[i, :] = acc
          return 0

        lax.fori_loop(0, CHUNK, row_body, 0, unroll=2)
        start_out(g, b)
      return 0

    lax.fori_loop(0, NCHUNK // 2, chunk_pair, 0)
    wait_out(0)
    wait_out(1)

  return k


def _make_dequant_kernel():
  mesh = plsc.VectorSubcoreMesh(core_axis_name="c", subcore_axis_name="s")

  @functools.partial(
      pl.kernel,
      mesh=mesh,
      out_type=jax.ShapeDtypeStruct((ROWS, EMB_DIM), jnp.float32),
      compiler_params=pltpu.CompilerParams(use_tc_tiling_on_sc=True),
      scratch_types=[
          pltpu.VMEM((CHUNK_WORDS,), jnp.int32),
          pltpu.VMEM((CHUNK_WORDS,), jnp.int32),
          pltpu.VMEM((CHUNK, EMB_DIM), jnp.float32),
          pltpu.VMEM((CHUNK, EMB_DIM), jnp.float32),
          pltpu.VMEM((L,), jnp.float32),
          pltpu.SemaphoreType.DMA,
          pltpu.SemaphoreType.DMA,
          pltpu.SemaphoreType.DMA,
          pltpu.SemaphoreType.DMA,
      ],
  )
  def k(g_hbm, c_hbm, out_hbm, gv0, gv1, ov0, ov1, c_v, sg0, sg1, so0, so1):
    wid = lax.axis_index("s") * NC + lax.axis_index("c")
    tbase = wid * ROWS_PER_W

    gv = (gv0, gv1)
    ov = (ov0, ov1)
    sg = (sg0, sg1)
    so = (so0, so1)

    pltpu.sync_copy(c_hbm, c_v)
    lut = (lax.iota(jnp.int32, L).astype(jnp.float32) - 7.0) / c_v[...]

    def vperm(src, idx):
      return lax.gather(
          src, idx[:, None],
          lax.GatherDimensionNumbers(
              offset_dims=(), collapsed_slice_dims=(0,),
              start_index_map=(0,)),
          slice_sizes=(1,),
          mode=lax.GatherScatterMode.PROMISE_IN_BOUNDS)

    def start_load(g, b):
      off = (tbase + g * CHUNK) * PACKED_WORDS
      pltpu.async_copy(g_hbm.at[pl.ds(off, CHUNK_WORDS)], gv[b], sg[b])

    def wait_load(b):
      pltpu.make_async_copy(
          g_hbm.at[pl.ds(0, CHUNK_WORDS)], gv[b], sg[b]).wait()

    def start_out(g, b):
      pltpu.async_copy(
          ov[b], out_hbm.at[pl.ds(tbase + g * CHUNK, CHUNK), :], so[b])

    def wait_out(b):
      pltpu.make_async_copy(
          ov[b], out_hbm.at[pl.ds(0, CHUNK), :], so[b]).wait()

    start_load(0, 0)

    def chunk_pair(g2, _):
      for b in range(2):
        g = 2 * g2 + b
        nxt = g + 1

        @pl.when(nxt < NCHUNK)
        def _():
          start_load(nxt, 1 - b)

        wait_load(b)

        @pl.when(g >= 2)
        def _():
          wait_out(b)

        gvb = gv[b]
        ovb = ov[b]

        def row_body(q, _):
          w = gvb[pl.ds(q * PACKED_WORDS, PACKED_WORDS)]
          for s in range(8):
            nib = lax.shift_right_logical(w, 4 * s) & 15
            ovb[q, pl.ds(s * L, L)] = vperm(lut, nib)
          return 0

        lax.fori_loop(0, CHUNK, row_body, 0, unroll=4)
        start_out(g, b)
      return 0

    lax.fori_loop(0, NCHUNK // 2, chunk_pair, 0)
    wait_out(0)
    wait_out(1)

  return k


_gather_kernel = _make_gather_kernel()
_dequant_kernel = _make_dequant_kernel()


@jax.jit
def kernel(x, weight_quant_packed, c):
  tab32 = lax.bitcast_convert_type(
      weight_quant_packed.reshape(NUM_EMBEDDINGS, PACKED_WORDS, 4), jnp.int32)
  idx = x.T.reshape(ROWS)
  c_vec = jnp.full((L,), c, dtype=jnp.float32)
  g = _gather_kernel(tab32, idx)
  out = _dequant_kernel(g.reshape(ROWS * PACKED_WORDS), c_vec)
  return out.reshape(FIELDS, BATCH, EMB_DIM).transpose(1, 0, 2)
